# baseline (speedup 1.0000x reference)
import functools

import jax
import jax.numpy as jnp
from jax import lax
from jax.experimental import pallas as pl
from jax.experimental.pallas import tpu as pltpu
from jax.experimental.pallas import tpu_sc as plsc

B = 16384
D = 128
NC = 2
NS = 16
NW = NC * NS
BPW = B // NW
CH = 128
NCH = BPW // CH
LANES = 16


def _prod_chunk(u_ref, i_ref, p_ref):
    def row(r, _):
        for k in range(D // LANES):
            sl = pl.ds(LANES * k, LANES)
            p_ref[r, sl] = u_ref[r, sl] * i_ref[r, sl]
        return 0

    lax.fori_loop(0, CH, row, 0, unroll=2)


def _gather_body(uid_ref, iid_ref, ug_t, ig_t, um_t, im_t,
                 pr_o, um_o, im_o,
                 uidx_v, iidx_v, bu, bi, bp, gsem, ssem):
    wid = lax.axis_index("s") * NC + lax.axis_index("c")
    base = wid * BPW
    pltpu.sync_copy(uid_ref.at[wid], uidx_v)
    pltpu.sync_copy(iid_ref.at[wid], iidx_v)

    pend = [pltpu.async_copy(ug_t.at[uidx_v.at[0]], bu.at[0], gsem),
            pltpu.async_copy(ig_t.at[iidx_v.at[0]], bi.at[0], gsem)]
    pend_s = [None, None]
    for c in range(NCH):
        s = c % 2
        nxt = None
        if c + 1 < NCH:
            nxt = [pltpu.async_copy(ug_t.at[uidx_v.at[c + 1]],
                                    bu.at[1 - s], gsem),
                   pltpu.async_copy(ig_t.at[iidx_v.at[c + 1]],
                                    bi.at[1 - s], gsem)]
        for h in pend:
            h.wait()
        if pend_s[s] is not None:
            pend_s[s].wait()
        _prod_chunk(bu.at[s], bi.at[s], bp.at[s])
        pend_s[s] = pltpu.async_copy(
            bp.at[s], pr_o.at[pl.ds(base + c * CH, CH)], ssem)
        pend = nxt

    pend = [pltpu.async_copy(um_t.at[uidx_v.at[0]], bu.at[0], gsem),
            pltpu.async_copy(im_t.at[iidx_v.at[0]], bi.at[0], gsem)]
    pend_s2 = [None, None]
    for c in range(NCH):
        s = c % 2
        nxt = None
        if c + 1 < NCH:
            if pend_s2[1 - s] is not None:
                for h in pend_s2[1 - s]:
                    h.wait()
                pend_s2[1 - s] = None
            nxt = [pltpu.async_copy(um_t.at[uidx_v.at[c + 1]],
                                    bu.at[1 - s], gsem),
                   pltpu.async_copy(im_t.at[iidx_v.at[c + 1]],
                                    bi.at[1 - s], gsem)]
        for h in pend:
            h.wait()
        if pend_s2[s] is not None:
            for h in pend_s2[s]:
                h.wait()
        pend_s2[s] = [
            pltpu.async_copy(bu.at[s], um_o.at[pl.ds(base + c * CH, CH)],
                             ssem),
            pltpu.async_copy(bi.at[s], im_o.at[pl.ds(base + c * CH, CH)],
                             ssem)]
        pend = nxt
    for hs in pend_s2:
        if hs is not None:
            for h in hs:
                h.wait()
    for h in pend_s:
        if h is not None:
            h.wait()


def _sc_gather(user_ids, item_ids, ue_gmf, ie_gmf, ue_mlp, ie_mlp):
    mesh = plsc.VectorSubcoreMesh(core_axis_name="c", subcore_axis_name="s",
                                  num_cores=NC, num_subcores=NS)
    f = pl.kernel(
        _gather_body,
        out_type=[jax.ShapeDtypeStruct((B, D), jnp.float32)] * 3,
        mesh=mesh,
        scratch_types=[
            pltpu.VMEM((NCH, CH), jnp.int32),
            pltpu.VMEM((NCH, CH), jnp.int32),
            pltpu.VMEM((2, CH, D), jnp.float32),
            pltpu.VMEM((2, CH, D), jnp.float32),
            pltpu.VMEM((2, CH, D), jnp.float32),
            pltpu.SemaphoreType.DMA,
            pltpu.SemaphoreType.DMA,
        ],
    )
    uid = user_ids.astype(jnp.int32).reshape(NW, NCH, CH)
    iid = item_ids.astype(jnp.int32).reshape(NW, NCH, CH)
    return f(uid, iid, ue_gmf, ie_gmf, ue_mlp, ie_mlp)


BB = 512


def _mlp_body(pr, um, im, w1u, w1i, b1, w2, b2, w3, b3, wog, woh, bo, out):
    dot = functools.partial(jnp.dot, preferred_element_type=jnp.float32)
    bf = jnp.bfloat16
    h = dot(um[...].astype(bf), w1u[...]) + dot(im[...].astype(bf), w1i[...])
    h = jnp.maximum(h + b1[...], 0.0)
    h = jnp.maximum(dot(h.astype(bf), w2[...]) + b2[...], 0.0)
    h = jnp.maximum(dot(h.astype(bf), w3[...]) + b3[...], 0.0)
    out[...] = (dot(pr[...].astype(bf), wog[...])
                + dot(h.astype(bf), woh[...]) + bo[0, 0])


def _tc_mlp(pr, um, im, W1, b1, W2, b2, W3, b3, Wo, bo):
    row = lambda i: (i, 0)
    zero = lambda i: (0, 0)
    rows_spec = pl.BlockSpec((BB, D), row)
    bf = jnp.bfloat16
    out = pl.pallas_call(
        _mlp_body,
        grid=(B // BB,),
        in_specs=[
            rows_spec, rows_spec, rows_spec,
            pl.BlockSpec((D, 256), zero),
            pl.BlockSpec((D, 256), zero),
            pl.BlockSpec((1, 256), zero),
            pl.BlockSpec((256, 128), zero),
            pl.BlockSpec((1, 128), zero),
            pl.BlockSpec((128, 64), zero),
            pl.BlockSpec((1, 64), zero),
            pl.BlockSpec((D, 1), zero),
            pl.BlockSpec((64, 1), zero),
            pl.BlockSpec((1, 1), zero),
        ],
        out_specs=pl.BlockSpec((BB, 1), row),
        out_shape=jax.ShapeDtypeStruct((B, 1), jnp.float32),
        compiler_params=pltpu.CompilerParams(
            dimension_semantics=("arbitrary",)),
    )(pr, um, im, W1[:D].astype(bf), W1[D:].astype(bf), b1.reshape(1, 256),
      W2.astype(bf), b2.reshape(1, 128), W3.astype(bf), b3.reshape(1, 64),
      Wo[:D].astype(bf), Wo[D:].astype(bf), bo.reshape(1, 1))
    return out[:, 0]


def kernel(user_ids, item_ids, ue_gmf, ie_gmf, ue_mlp, ie_mlp,
           W1, b1, W2, b2, W3, b3, Wo, bo):
    pr, um, im = _sc_gather(user_ids, item_ids, ue_gmf, ie_gmf,
                            ue_mlp, ie_mlp)
    return _tc_mlp(pr, um, im, W1, b1, W2, b2, W3, b3, Wo, bo)


# speedup vs baseline: 1.1877x; 1.1877x over previous
import functools

import jax
import jax.numpy as jnp
from jax import lax
from jax.experimental import pallas as pl
from jax.experimental.pallas import tpu as pltpu
from jax.experimental.pallas import tpu_sc as plsc

B = 16384
D = 128
NC = 2
NS = 16
NW = NC * NS
BPW = B // NW
CH = 128
NCH = BPW // CH
LANES = 16


def _prod_chunk(u_ref, i_ref, p_ref):
    def row(r, _):
        for k in range(D // LANES):
            sl = pl.ds(LANES * k, LANES)
            p_ref[r, sl] = u_ref[r, sl] * i_ref[r, sl]
        return 0

    lax.fori_loop(0, CH, row, 0, unroll=2)


def _gather_body(uid_ref, iid_ref, ug_t, ig_t, um_t, im_t,
                 pr_o, um_o, im_o,
                 uidx_v, iidx_v, bu, bi, bp, gsem, ssem):
    wid = lax.axis_index("s") * NC + lax.axis_index("c")
    base = wid * BPW
    pltpu.sync_copy(uid_ref.at[wid], uidx_v)
    pltpu.sync_copy(iid_ref.at[wid], iidx_v)

    pend = [pltpu.async_copy(ug_t.at[uidx_v.at[0]], bu.at[0], gsem),
            pltpu.async_copy(ig_t.at[iidx_v.at[0]], bi.at[0], gsem)]
    pend_s = [None, None]
    for c in range(NCH):
        s = c % 2
        nxt = None
        if c + 1 < NCH:
            nxt = [pltpu.async_copy(ug_t.at[uidx_v.at[c + 1]],
                                    bu.at[1 - s], gsem),
                   pltpu.async_copy(ig_t.at[iidx_v.at[c + 1]],
                                    bi.at[1 - s], gsem)]
        for h in pend:
            h.wait()
        if pend_s[s] is not None:
            pend_s[s].wait()
        _prod_chunk(bu.at[s], bi.at[s], bp.at[s])
        pend_s[s] = pltpu.async_copy(
            bp.at[s], pr_o.at[pl.ds(base + c * CH, CH)], ssem)
        pend = nxt

    pend = [pltpu.async_copy(um_t.at[uidx_v.at[0]], bu.at[0], gsem),
            pltpu.async_copy(im_t.at[iidx_v.at[0]], bi.at[0], gsem)]
    pend_s2 = [None, None]
    for c in range(NCH):
        s = c % 2
        nxt = None
        if c + 1 < NCH:
            if pend_s2[1 - s] is not None:
                for h in pend_s2[1 - s]:
                    h.wait()
                pend_s2[1 - s] = None
            nxt = [pltpu.async_copy(um_t.at[uidx_v.at[c + 1]],
                                    bu.at[1 - s], gsem),
                   pltpu.async_copy(im_t.at[iidx_v.at[c + 1]],
                                    bi.at[1 - s], gsem)]
        for h in pend:
            h.wait()
        if pend_s2[s] is not None:
            for h in pend_s2[s]:
                h.wait()
        pend_s2[s] = [
            pltpu.async_copy(bu.at[s], um_o.at[pl.ds(base + c * CH, CH)],
                             ssem),
            pltpu.async_copy(bi.at[s], im_o.at[pl.ds(base + c * CH, CH)],
                             ssem)]
        pend = nxt
    for hs in pend_s2:
        if hs is not None:
            for h in hs:
                h.wait()
    for h in pend_s:
        if h is not None:
            h.wait()


def _sc_gather(user_ids, item_ids, ue_gmf, ie_gmf, ue_mlp, ie_mlp):
    mesh = plsc.VectorSubcoreMesh(core_axis_name="c", subcore_axis_name="s",
                                  num_cores=NC, num_subcores=NS)
    f = pl.kernel(
        _gather_body,
        out_type=[jax.ShapeDtypeStruct((B, D), jnp.float32)] * 3,
        mesh=mesh,
        scratch_types=[
            pltpu.VMEM((NCH, CH), jnp.int32),
            pltpu.VMEM((NCH, CH), jnp.int32),
            pltpu.VMEM((2, CH, D), jnp.float32),
            pltpu.VMEM((2, CH, D), jnp.float32),
            pltpu.VMEM((2, CH, D), jnp.float32),
            pltpu.SemaphoreType.DMA,
            pltpu.SemaphoreType.DMA,
        ],
    )
    uid = user_ids.astype(jnp.int32).reshape(NW, NCH, CH)
    iid = item_ids.astype(jnp.int32).reshape(NW, NCH, CH)
    return f(uid, iid, ue_gmf, ie_gmf, ue_mlp, ie_mlp)


BB = 2048


def _mlp_body(pr, um, im, w1, b1, w2, b2, w3, b3, wo, bo, out):
    dot = functools.partial(jnp.dot, preferred_element_type=jnp.float32)
    bf = jnp.bfloat16
    w1b = w1[...].astype(bf)
    h = dot(um[...].astype(bf), w1b[:D]) + dot(im[...].astype(bf), w1b[D:])
    h = jnp.maximum(h + b1[...], 0.0)
    h = jnp.maximum(dot(h.astype(bf), w2[...].astype(bf)) + b2[...], 0.0)
    h = jnp.maximum(dot(h.astype(bf), w3[...].astype(bf)) + b3[...], 0.0)
    wob = wo[...].astype(bf)
    out[...] = (dot(pr[...].astype(bf), wob[:D])
                + dot(h.astype(bf), wob[D:]) + bo[0, 0])


def _tc_mlp(pr, um, im, W1, b1, W2, b2, W3, b3, Wo, bo):
    row = lambda i: (i, 0)
    zero = lambda i: (0, 0)
    rows_spec = pl.BlockSpec((BB, D), row)
    out = pl.pallas_call(
        _mlp_body,
        grid=(B // BB,),
        in_specs=[
            rows_spec, rows_spec, rows_spec,
            pl.BlockSpec((256, 256), zero),
            pl.BlockSpec((1, 256), zero),
            pl.BlockSpec((256, 128), zero),
            pl.BlockSpec((1, 128), zero),
            pl.BlockSpec((128, 64), zero),
            pl.BlockSpec((1, 64), zero),
            pl.BlockSpec((192, 1), zero),
            pl.BlockSpec((1, 1), zero),
        ],
        out_specs=pl.BlockSpec((BB, 1), row),
        out_shape=jax.ShapeDtypeStruct((B, 1), jnp.float32),
        compiler_params=pltpu.CompilerParams(
            dimension_semantics=("arbitrary",)),
    )(pr, um, im, W1, b1.reshape(1, 256), W2, b2.reshape(1, 128), W3,
      b3.reshape(1, 64), Wo, bo.reshape(1, 1))
    return out[:, 0]


def kernel(user_ids, item_ids, ue_gmf, ie_gmf, ue_mlp, ie_mlp,
           W1, b1, W2, b2, W3, b3, Wo, bo):
    pr, um, im = _sc_gather(user_ids, item_ids, ue_gmf, ie_gmf,
                            ue_mlp, ie_mlp)
    return _tc_mlp(pr, um, im, W1, b1, W2, b2, W3, b3, Wo, bo)


# speedup vs baseline: 1.2231x; 1.0298x over previous
import functools

import jax
import jax.numpy as jnp
from jax import lax
from jax.experimental import pallas as pl
from jax.experimental.pallas import tpu as pltpu
from jax.experimental.pallas import tpu_sc as plsc

B = 16384
D = 128
NC = 2
NS = 16
NW = NC * NS
BPW = B // NW
CH = 128
NCH = BPW // CH
LANES = 16


def _prod_chunk(u_ref, i_ref, p_ref):
    def row(r, _):
        for k in range(D // LANES):
            sl = pl.ds(LANES * k, LANES)
            p_ref[r, sl] = u_ref[r, sl] * i_ref[r, sl]
        return 0

    lax.fori_loop(0, CH, row, 0, unroll=2)


def _make_gather_body(nch):
    bpw = nch * CH
    return functools.partial(_gather_body_impl, nch, bpw)


def _gather_body_impl(NCH, BPW, uid_ref, iid_ref, ug_t, ig_t, um_t, im_t,
                      pr_o, um_o, im_o,
                      uidx_v, iidx_v, bu, bi, bp, gsem, ssem):
    wid = lax.axis_index("s") * NC + lax.axis_index("c")
    base = wid * BPW
    pltpu.sync_copy(uid_ref.at[wid], uidx_v)
    pltpu.sync_copy(iid_ref.at[wid], iidx_v)

    pend = [pltpu.async_copy(ug_t.at[uidx_v.at[0]], bu.at[0], gsem),
            pltpu.async_copy(ig_t.at[iidx_v.at[0]], bi.at[0], gsem)]
    pend_s = [None, None]
    for c in range(NCH):
        s = c % 2
        nxt = None
        if c + 1 < NCH:
            nxt = [pltpu.async_copy(ug_t.at[uidx_v.at[c + 1]],
                                    bu.at[1 - s], gsem),
                   pltpu.async_copy(ig_t.at[iidx_v.at[c + 1]],
                                    bi.at[1 - s], gsem)]
        for h in pend:
            h.wait()
        if pend_s[s] is not None:
            pend_s[s].wait()
        _prod_chunk(bu.at[s], bi.at[s], bp.at[s])
        pend_s[s] = pltpu.async_copy(
            bp.at[s], pr_o.at[pl.ds(base + c * CH, CH)], ssem)
        pend = nxt

    pend = [pltpu.async_copy(um_t.at[uidx_v.at[0]], bu.at[0], gsem),
            pltpu.async_copy(im_t.at[iidx_v.at[0]], bi.at[0], gsem)]
    pend_s2 = [None, None]
    for c in range(NCH):
        s = c % 2
        nxt = None
        if c + 1 < NCH:
            if pend_s2[1 - s] is not None:
                for h in pend_s2[1 - s]:
                    h.wait()
                pend_s2[1 - s] = None
            nxt = [pltpu.async_copy(um_t.at[uidx_v.at[c + 1]],
                                    bu.at[1 - s], gsem),
                   pltpu.async_copy(im_t.at[iidx_v.at[c + 1]],
                                    bi.at[1 - s], gsem)]
        for h in pend:
            h.wait()
        if pend_s2[s] is not None:
            for h in pend_s2[s]:
                h.wait()
        pend_s2[s] = [
            pltpu.async_copy(bu.at[s], um_o.at[pl.ds(base + c * CH, CH)],
                             ssem),
            pltpu.async_copy(bi.at[s], im_o.at[pl.ds(base + c * CH, CH)],
                             ssem)]
        pend = nxt
    for hs in pend_s2:
        if hs is not None:
            for h in hs:
                h.wait()
    for h in pend_s:
        if h is not None:
            h.wait()


def _sc_gather(user_ids, item_ids, ue_gmf, ie_gmf, ue_mlp, ie_mlp):
    nb = user_ids.shape[0]
    nch = nb // (NW * CH)
    mesh = plsc.VectorSubcoreMesh(core_axis_name="c", subcore_axis_name="s",
                                  num_cores=NC, num_subcores=NS)
    f = pl.kernel(
        _make_gather_body(nch),
        out_type=[jax.ShapeDtypeStruct((nb, D), jnp.float32)] * 3,
        mesh=mesh,
        scratch_types=[
            pltpu.VMEM((nch, CH), jnp.int32),
            pltpu.VMEM((nch, CH), jnp.int32),
            pltpu.VMEM((2, CH, D), jnp.float32),
            pltpu.VMEM((2, CH, D), jnp.float32),
            pltpu.VMEM((2, CH, D), jnp.float32),
            pltpu.SemaphoreType.DMA,
            pltpu.SemaphoreType.DMA,
        ],
    )
    uid = user_ids.astype(jnp.int32).reshape(NW, nch, CH)
    iid = item_ids.astype(jnp.int32).reshape(NW, nch, CH)
    return f(uid, iid, ue_gmf, ie_gmf, ue_mlp, ie_mlp)


BB = 2048


def _mlp_body(pr, um, im, w1, b1, w2, b2, w3, b3, wo, bo, out):
    dot = functools.partial(jnp.dot, preferred_element_type=jnp.float32)
    bf = jnp.bfloat16
    w1b = w1[...].astype(bf)
    h = dot(um[...].astype(bf), w1b[:D]) + dot(im[...].astype(bf), w1b[D:])
    h = jnp.maximum(h + b1[...], 0.0)
    h = jnp.maximum(dot(h.astype(bf), w2[...].astype(bf)) + b2[...], 0.0)
    h = jnp.maximum(dot(h.astype(bf), w3[...].astype(bf)) + b3[...], 0.0)
    wob = wo[...].astype(bf)
    out[...] = (dot(pr[...].astype(bf), wob[:D])
                + dot(h.astype(bf), wob[D:]) + bo[0, 0])


def _tc_mlp(pr, um, im, W1, b1, W2, b2, W3, b3, Wo, bo):
    row = lambda i: (i, 0)
    zero = lambda i: (0, 0)
    nb = pr.shape[0]
    rows_spec = pl.BlockSpec((BB, D), row)
    out = pl.pallas_call(
        _mlp_body,
        grid=(nb // BB,),
        in_specs=[
            rows_spec, rows_spec, rows_spec,
            pl.BlockSpec((256, 256), zero),
            pl.BlockSpec((1, 256), zero),
            pl.BlockSpec((256, 128), zero),
            pl.BlockSpec((1, 128), zero),
            pl.BlockSpec((128, 64), zero),
            pl.BlockSpec((1, 64), zero),
            pl.BlockSpec((192, 1), zero),
            pl.BlockSpec((1, 1), zero),
        ],
        out_specs=pl.BlockSpec((BB, 1), row),
        out_shape=jax.ShapeDtypeStruct((nb, 1), jnp.float32),
        compiler_params=pltpu.CompilerParams(
            dimension_semantics=("arbitrary",)),
    )(pr, um, im, W1, b1.reshape(1, 256), W2, b2.reshape(1, 128), W3,
      b3.reshape(1, 64), Wo, bo.reshape(1, 1))
    return out[:, 0]


NSPLIT = 2


def kernel(user_ids, item_ids, ue_gmf, ie_gmf, ue_mlp, ie_mlp,
           W1, b1, W2, b2, W3, b3, Wo, bo):
    h = B // NSPLIT
    outs = []
    for k in range(NSPLIT):
        pr, um, im = _sc_gather(user_ids[k * h:(k + 1) * h],
                                item_ids[k * h:(k + 1) * h],
                                ue_gmf, ie_gmf, ue_mlp, ie_mlp)
        outs.append(_tc_mlp(pr, um, im, W1, b1, W2, b2, W3, b3, Wo, bo))
    return jnp.concatenate(outs) if NSPLIT > 1 else outs[0]
